# baseline (speedup 1.0000x reference)
import functools

import jax
import jax.numpy as jnp
from jax import lax
from jax.experimental import pallas as pl
from jax.experimental.pallas import tpu as pltpu
from jax.experimental.pallas import tpu_sc as plsc

_VOCAB = 1_000_000
_DIM = 32
_BATCH = 16384
_HIST = 50

_NC = 2
_NS = 16
_NW = _NC * _NS
_BW = _BATCH // _NW
_CHUNK = 128
_NCHUNK = _BW // _CHUNK

_mesh = plsc.VectorSubcoreMesh(core_axis_name="c", subcore_axis_name="s")


@functools.partial(
    pl.kernel,
    out_type=jax.ShapeDtypeStruct((_HIST, 4, 128, 8, 128), jnp.float32),
    mesh=_mesh,
    scratch_types=[
        pltpu.VMEM((_HIST, _NCHUNK, _CHUNK), jnp.int32),
        pltpu.VMEM((2, _BW, _DIM), jnp.float32),
        pltpu.VMEM((2, 4, _NCHUNK, 8, 128), jnp.float32),
        pltpu.SemaphoreType.DMA,
        pltpu.SemaphoreType.DMA,
    ],
    compiler_params=pltpu.CompilerParams(
        use_tc_tiling_on_sc=False,
        needs_layout_passes=False,
        disable_bounds_checks=True,
    ),
)
def _gather_kernel(idx_hbm, table_hbm, out_hbm, idx_all, rows_v, tbuf,
                   gsem, wsem):
    wid = lax.axis_index("s") * _NC + lax.axis_index("c")
    lane = lax.iota(jnp.int32, 16)

    pltpu.sync_copy(idx_hbm.at[:, pl.ds(wid * _NCHUNK, _NCHUNK)], idx_all)

    def fire_gathers(h, slot):
        for j in range(_NCHUNK):
            pltpu.async_copy(
                table_hbm.at[idx_all.at[h, j]],
                rows_v.at[slot, pl.ds(j * _CHUNK, _CHUNK)],
                gsem,
            )

    def drain_gathers(slot):
        pltpu.make_async_copy(
            table_hbm.at[pl.ds(0, _BW)], rows_v.at[slot], gsem
        ).wait()

    def drain_writes(slot):
        for tr in range(4):
            pltpu.make_async_copy(
                tbuf.at[slot, tr], out_hbm.at[0, tr, pl.ds(0, _NCHUNK)], wsem
            ).wait()

    fire_gathers(0, 0)
    fire_gathers(1, 1)

    def group(g, _):
        for b in range(2):
            h = g * 2 + b
            drain_gathers(b)

            @pl.when(g >= 1)
            def _():
                drain_writes(b)

            @plsc.parallel_loop(0, 32, unroll=8)
            def per_q(q):
                tr = q // 8
                sl = q - tr * 8
                col = jnp.zeros((16,), jnp.int32) + q
                for tc in range(_NCHUNK):
                    for l0 in range(0, 128, 16):
                        row = jnp.full((16,), tc * 128 + l0, jnp.int32) + lane
                        tbuf[b, tr, tc, sl, pl.ds(l0, 16)] = (
                            plsc.load_gather(rows_v.at[b], [row, col])
                        )

            for tr in range(4):
                pltpu.async_copy(
                    tbuf.at[b, tr],
                    out_hbm.at[h, tr, pl.ds(wid * _NCHUNK, _NCHUNK)],
                    wsem,
                )

            @pl.when(g < _HIST // 2 - 1)
            def _():
                fire_gathers(h + 2, b)

        return 0

    lax.fori_loop(0, _HIST // 2, group, 0)
    for b in range(2):
        drain_writes(b)


def kernel(indices, table):
    idx_t = (indices.T * 4).reshape(_HIST, _BATCH // 128, 128)
    tblp = jnp.pad(table, ((0, 0), (0, 96))).reshape(4 * _VOCAB, _DIM)
    out5 = _gather_kernel(idx_t, tblp)
    return out5.transpose(2, 4, 0, 1, 3).reshape(_BATCH, _HIST, _DIM)


# speedup vs baseline: 1.0239x; 1.0239x over previous
import functools

import jax
import jax.numpy as jnp
from jax import lax
from jax.experimental import pallas as pl
from jax.experimental.pallas import tpu as pltpu
from jax.experimental.pallas import tpu_sc as plsc

_VOCAB = 1_000_000
_DIM = 32
_BATCH = 16384
_HIST = 50

_NC = 2
_NS = 16
_NW = _NC * _NS
_BW = _BATCH // _NW
_CHUNK = 128
_NCHUNK = _BW // _CHUNK

_mesh = plsc.VectorSubcoreMesh(core_axis_name="c", subcore_axis_name="s")


@functools.partial(
    pl.kernel,
    out_type=jax.ShapeDtypeStruct((_HIST, 4, 128, 8, 128), jnp.float32),
    mesh=_mesh,
    scratch_types=[
        pltpu.VMEM((_HIST, _NCHUNK, _CHUNK), jnp.int32),
        pltpu.VMEM((2, _BW, _DIM), jnp.float32),
        pltpu.VMEM((2, 4, _NCHUNK, 8, 128), jnp.float32),
        pltpu.SemaphoreType.DMA,
        pltpu.SemaphoreType.DMA,
    ],
    compiler_params=pltpu.CompilerParams(
        use_tc_tiling_on_sc=False,
        needs_layout_passes=False,
        disable_bounds_checks=True,
    ),
)
def _gather_kernel(idx_hbm, table_hbm, out_hbm, idx_all, rows_v, tbuf,
                   gsem, wsem):
    wid = lax.axis_index("s") * _NC + lax.axis_index("c")
    lane = lax.iota(jnp.int32, 16)

    pltpu.sync_copy(idx_hbm.at[:, pl.ds(wid * _NCHUNK, _NCHUNK)], idx_all)

    def fire_gathers(h, slot):
        for j in range(_NCHUNK):
            pltpu.async_copy(
                table_hbm.at[idx_all.at[h, j]],
                rows_v.at[slot, pl.ds(j * _CHUNK, _CHUNK)],
                gsem,
            )

    def drain_gathers(slot):
        pltpu.make_async_copy(
            table_hbm.at[pl.ds(0, _BW)], rows_v.at[slot], gsem
        ).wait()

    def drain_writes(slot):
        for tr in range(4):
            pltpu.make_async_copy(
                tbuf.at[slot, tr], out_hbm.at[0, tr, pl.ds(0, _NCHUNK)], wsem
            ).wait()

    fire_gathers(0, 0)
    fire_gathers(1, 1)

    def group(g, _):
        for b in range(2):
            h = g * 2 + b
            drain_gathers(b)

            @pl.when(g >= 1)
            def _():
                drain_writes(b)

            @plsc.parallel_loop(0, 128, unroll=4)
            def per_qtc(p):
                q = p // _NCHUNK
                tc = p - q * _NCHUNK
                tr = q // 8
                sl = q - tr * 8
                col = jnp.zeros((16,), jnp.int32) + q
                for l0 in range(0, 128, 16):
                    row = jnp.full((16,), l0, jnp.int32) + lane + tc * 128
                    tbuf[b, tr, tc, sl, pl.ds(l0, 16)] = (
                        plsc.load_gather(rows_v.at[b], [row, col])
                    )

            for tr in range(4):
                pltpu.async_copy(
                    tbuf.at[b, tr],
                    out_hbm.at[h, tr, pl.ds(wid * _NCHUNK, _NCHUNK)],
                    wsem,
                )

            @pl.when(g < _HIST // 2 - 1)
            def _():
                fire_gathers(h + 2, b)

        return 0

    lax.fori_loop(0, _HIST // 2, group, 0)
    for b in range(2):
        drain_writes(b)


def kernel(indices, table):
    idx_t = (indices.T * 4).reshape(_HIST, _BATCH // 128, 128)
    tblp = jnp.pad(table, ((0, 0), (0, 96))).reshape(4 * _VOCAB, _DIM)
    out5 = _gather_kernel(idx_t, tblp)
    return out5.transpose(2, 4, 0, 1, 3).reshape(_BATCH, _HIST, _DIM)


# speedup vs baseline: 1.0815x; 1.0563x over previous
import functools

import jax
import jax.numpy as jnp
from jax import lax
from jax.experimental import pallas as pl
from jax.experimental.pallas import tpu as pltpu
from jax.experimental.pallas import tpu_sc as plsc

_VOCAB = 1_000_000
_DIM = 32
_BATCH = 16384
_HIST = 50

_NC = 2
_NS = 16
_NW = _NC * _NS
_BW = _BATCH // _NW
_CHUNK = 128
_NCHUNK = _BW // _CHUNK

_mesh = plsc.VectorSubcoreMesh(core_axis_name="c", subcore_axis_name="s")


@functools.partial(
    pl.kernel,
    out_type=jax.ShapeDtypeStruct((_HIST, 4, 128, 8, 128), jnp.float32),
    mesh=_mesh,
    scratch_types=[
        pltpu.VMEM((_HIST, _NCHUNK, _CHUNK), jnp.int32),
        pltpu.VMEM((2, _BW, _DIM), jnp.float32),
        pltpu.VMEM((2, 4, _NCHUNK, 8, 128), jnp.float32),
        pltpu.SemaphoreType.DMA,
        pltpu.SemaphoreType.DMA,
    ],
    compiler_params=pltpu.CompilerParams(
        use_tc_tiling_on_sc=False,
        needs_layout_passes=False,
        disable_bounds_checks=True,
    ),
)
def _gather_kernel(idx_hbm, table_hbm, out_hbm, idx_all, rows_v, tbuf,
                   gsem, wsem):
    wid = lax.axis_index("s") * _NC + lax.axis_index("c")
    lane = lax.iota(jnp.int32, 16)

    pltpu.sync_copy(idx_hbm.at[:, pl.ds(wid * _NCHUNK, _NCHUNK)], idx_all)

    def fire_gathers(h, slot):
        for j in range(_NCHUNK):
            pltpu.async_copy(
                table_hbm.at[idx_all.at[h, j]],
                rows_v.at[slot, pl.ds(j * _CHUNK, _CHUNK)],
                gsem,
            )

    def drain_gathers(slot):
        pltpu.make_async_copy(
            table_hbm.at[pl.ds(0, _BW)], rows_v.at[slot], gsem
        ).wait()

    def drain_writes(slot):
        for tr in range(4):
            pltpu.make_async_copy(
                tbuf.at[slot, tr], out_hbm.at[0, tr, pl.ds(0, _NCHUNK)], wsem
            ).wait()

    fire_gathers(0, 0)
    fire_gathers(1, 1)

    def group(g, _):
        for b in range(2):
            h = g * 2 + b
            drain_gathers(b)

            @pl.when(g >= 1)
            def _():
                drain_writes(b)

            @plsc.parallel_loop(0, 32, unroll=2)
            def per_q(q):
                tr = q // 8
                sl = q - tr * 8
                col = jnp.zeros((16,), jnp.int32) + q
                for tc in range(_NCHUNK):
                    for l0 in range(0, 128, 16):
                        row = jnp.full((16,), tc * 128 + l0, jnp.int32) + lane
                        tbuf[b, tr, tc, sl, pl.ds(l0, 16)] = (
                            plsc.load_gather(rows_v.at[b], [row, col])
                        )

            for tr in range(4):
                pltpu.async_copy(
                    tbuf.at[b, tr],
                    out_hbm.at[h, tr, pl.ds(wid * _NCHUNK, _NCHUNK)],
                    wsem,
                )

            @pl.when(g < _HIST // 2 - 1)
            def _():
                fire_gathers(h + 2, b)

        return 0

    lax.fori_loop(0, _HIST // 2, group, 0)
    for b in range(2):
        drain_writes(b)


def kernel(indices, table):
    idx_t = (indices.T * 4).reshape(_HIST, _BATCH // 128, 128)
    tblp = jnp.pad(table, ((0, 0), (0, 96))).reshape(4 * _VOCAB, _DIM)
    out5 = _gather_kernel(idx_t, tblp)
    return out5.transpose(2, 4, 0, 1, 3).reshape(_BATCH, _HIST, _DIM)
